# baseline (speedup 1.0000x reference)
import jax
import jax.numpy as jnp
from jax import lax
from jax.experimental import pallas as pl
from jax.experimental.pallas import tpu as pltpu
from jax.experimental.pallas import tpu_sc as plsc

N, E, H, G = 10000, 160000, 256, 16
NT = 32
RPT = 320
NP = NT * RPT
CAP = 8064
KB = 48
ECH = 2000
NBLK = 10
BLK = NP // NBLK


def _iota16():
    return lax.iota(jnp.int32, 16)


def _lane_take(v, lane):
    idx = jnp.full((16,), lane, jnp.int32)
    return v.at[idx].get(mode="promise_in_bounds")


def _rsqrt16(x):
    i = plsc.bitcast(x, jnp.int32)
    y = plsc.bitcast(jnp.int32(0x5F3759DF) - (i >> 1), jnp.float32)
    for _ in range(4):
        y = y * (1.5 - 0.5 * x * y * y)
    return y


def _sc_prep_body(row_h, col_h, ew_h, srcl_h, dstl_h, ewl_h, cnt_h, dinv_h,
                  dsq_h, src_a, dst_a, ew_a, src_b, dst_b, ew_b, lsrc, ldst,
                  lew, deg_v, div_v, dsq_v, csc_v, sem_a, sem_b):
    wid = lax.axis_index("s") * 2 + lax.axis_index("c")
    base = wid * RPT
    zi = jnp.zeros((16,), jnp.int32)
    zf = jnp.zeros((16,), jnp.float32)
    nch = E // ECH

    def issue(b, bufs, sem):
        off = pl.multiple_of(b * ECH, 8)
        pltpu.async_copy(row_h.at[pl.ds(off, ECH)], bufs[0], sem)
        pltpu.async_copy(col_h.at[pl.ds(off, ECH)], bufs[1], sem)
        pltpu.async_copy(ew_h.at[pl.ds(off, ECH)], bufs[2], sem)

    def drain(bufs, sem):
        pltpu.make_async_copy(row_h.at[pl.ds(0, ECH)], bufs[0], sem).wait()
        pltpu.make_async_copy(row_h.at[pl.ds(0, ECH)], bufs[1], sem).wait()
        pltpu.make_async_copy(ew_h.at[pl.ds(0, ECH)], bufs[2], sem).wait()

    def compact(bufs, cnt):
        src_v, dst_v, ew_v = bufs
        GPB = 5

        def grp(i, cnt):
            gs = []
            for j in range(GPB):
                o = pl.ds((i * GPB + j) * 16, 16)
                sv = src_v[o]
                dv = dst_v[o]
                ev = ew_v[o]
                m = (dv >= base) & (dv < base + RPT)
                pos = plsc.cumsum(jnp.where(m, 1, 0).astype(jnp.int32))
                gs.append((sv, dv, ev, m, pos))
            for sv, dv, ev, m, pos in gs:
                offs = cnt + pos - 1
                mj = m & (offs < CAP - KB)
                plsc.store_scatter(lsrc, [offs], sv, mask=mj)
                plsc.store_scatter(ldst, [offs], dv - base, mask=mj)
                plsc.store_scatter(lew, [offs], ev, mask=mj)
                cnt = cnt + _lane_take(pos, 15)
            return cnt

        return lax.fori_loop(0, ECH // 16 // GPB, grp, cnt)

    bufs_a = (src_a, dst_a, ew_a)
    bufs_b = (src_b, dst_b, ew_b)
    issue(0, bufs_a, sem_a)

    def pair(p, cnt):
        b0 = 2 * p

        @pl.when(b0 + 1 < nch)
        def _():
            issue(b0 + 1, bufs_b, sem_b)

        drain(bufs_a, sem_a)
        cnt = compact(bufs_a, cnt)

        @pl.when(b0 + 2 < nch)
        def _():
            issue(b0 + 2, bufs_a, sem_a)

        drain(bufs_b, sem_b)
        cnt = compact(bufs_b, cnt)
        return cnt

    cnt_v16 = lax.fori_loop(0, nch // 2, pair, jnp.zeros((16,), jnp.int32))
    cnt = jnp.minimum(cnt_v16[0], jnp.int32(CAP - KB))

    for t in range(KB // 16):
        lsrc[pl.ds(cnt + t * 16, 16)] = zi
        ldst[pl.ds(cnt + t * 16, 16)] = zi
        lew[pl.ds(cnt + t * 16, 16)] = zf

    for t in range(RPT // 16):
        deg_v[pl.ds(t * 16, 16)] = zf

    def degg(g, carry):
        dlv = ldst[pl.ds(g * 16, 16)]
        ewv = lew[pl.ds(g * 16, 16)]
        for k in range(16):
            mk = _iota16() == k
            plsc.addupdate_scatter(deg_v, [dlv], ewv, mask=mk)
        return carry

    ng = (cnt + 15) // 16
    lax.fori_loop(0, ng, degg, jnp.int32(0))

    for t in range(RPT // 16):
        gidx = base + t * 16 + _iota16()
        d = deg_v[pl.ds(t * 16, 16)]
        d = d + jnp.where(gidx < N, 1.0, 0.0).astype(jnp.float32)
        r = _rsqrt16(jnp.maximum(d, 1e-12))
        r = jnp.where(d > 0, r, 0.0).astype(jnp.float32)
        div_v[pl.ds(t * 16, 16)] = r
        dsq_v[pl.ds(t * 16, 16)] = r * r

    csc_v[...] = jnp.full((16,), cnt, jnp.int32)
    pltpu.sync_copy(lsrc, srcl_h.at[wid])
    pltpu.sync_copy(ldst, dstl_h.at[wid])
    pltpu.sync_copy(lew, ewl_h.at[wid])
    pltpu.sync_copy(csc_v, cnt_h.at[wid])
    pltpu.sync_copy(div_v, dinv_h.at[pl.ds(base, RPT)])
    pltpu.sync_copy(dsq_v, dsq_h.at[pl.ds(base, RPT)])


def _sc_prep(row, col, ew):
    mesh = plsc.VectorSubcoreMesh(core_axis_name="c", subcore_axis_name="s")
    f = pl.kernel(
        _sc_prep_body,
        out_type=(
            jax.ShapeDtypeStruct((NT, CAP), jnp.int32),
            jax.ShapeDtypeStruct((NT, CAP), jnp.int32),
            jax.ShapeDtypeStruct((NT, CAP), jnp.float32),
            jax.ShapeDtypeStruct((NT, 16), jnp.int32),
            jax.ShapeDtypeStruct((NP,), jnp.float32),
            jax.ShapeDtypeStruct((NP,), jnp.float32),
        ),
        mesh=mesh,
        compiler_params=pltpu.CompilerParams(needs_layout_passes=False),
        scratch_types=[
            pltpu.VMEM((ECH,), jnp.int32),
            pltpu.VMEM((ECH,), jnp.int32),
            pltpu.VMEM((ECH,), jnp.float32),
            pltpu.VMEM((ECH,), jnp.int32),
            pltpu.VMEM((ECH,), jnp.int32),
            pltpu.VMEM((ECH,), jnp.float32),
            pltpu.VMEM((CAP,), jnp.int32),
            pltpu.VMEM((CAP,), jnp.int32),
            pltpu.VMEM((CAP,), jnp.float32),
            pltpu.VMEM((RPT,), jnp.float32),
            pltpu.VMEM((RPT,), jnp.float32),
            pltpu.VMEM((RPT,), jnp.float32),
            pltpu.VMEM((16,), jnp.int32),
            pltpu.SemaphoreType.DMA,
            pltpu.SemaphoreType.DMA,
        ],
    )
    return f(row, col, ew)


def _sc_norm_body(dinv_h, srcl_h, dstl_h, ewl_h, cnt_h, norml_h,
                  dinv_v, lsrc, ldst, lew, lnorm, csc_v):
    wid = lax.axis_index("s") * 2 + lax.axis_index("c")
    base = wid * RPT
    pltpu.sync_copy(dinv_h, dinv_v)
    pltpu.sync_copy(srcl_h.at[wid], lsrc)
    pltpu.sync_copy(dstl_h.at[wid], ldst)
    pltpu.sync_copy(ewl_h.at[wid], lew)
    pltpu.sync_copy(cnt_h.at[wid], csc_v)
    n = csc_v[pl.ds(0, 16)][0]
    ng = ((n + KB - 1) // KB) * (KB // 16)

    def grp(g, carry):
        sv = lsrc[pl.ds(g * 16, 16)]
        dlv = ldst[pl.ds(g * 16, 16)]
        ev = lew[pl.ds(g * 16, 16)]
        da = plsc.load_gather(dinv_v, [sv])
        db = plsc.load_gather(dinv_v, [dlv + base])
        lnorm[pl.ds(g * 16, 16)] = da * ev * db
        return carry

    lax.fori_loop(0, ng, grp, jnp.int32(0))
    pltpu.sync_copy(lnorm, norml_h.at[wid])


def _sc_norm(dinv, srcl, dstl, ewl, cnt):
    mesh = plsc.VectorSubcoreMesh(core_axis_name="c", subcore_axis_name="s")
    f = pl.kernel(
        _sc_norm_body,
        out_type=jax.ShapeDtypeStruct((NT, CAP), jnp.float32),
        mesh=mesh,
        compiler_params=pltpu.CompilerParams(needs_layout_passes=False),
        scratch_types=[
            pltpu.VMEM((NP,), jnp.float32),
            pltpu.VMEM((CAP,), jnp.int32),
            pltpu.VMEM((CAP,), jnp.int32),
            pltpu.VMEM((CAP,), jnp.float32),
            pltpu.VMEM((CAP,), jnp.float32),
            pltpu.VMEM((16,), jnp.int32),
        ],
    )
    return f(dinv, srcl, dstl, ewl, cnt)


def _sc_agg_body(xw_h, srcl_h, dstl_h, norml_h, cnt_h, agg_h,
                 acc, rows0, rows1, rows2, rows3, lsrc, ldst, lnrm, csc_v,
                 sem0, sem1, sem2, sem3):
    wid = lax.axis_index("s") * 2 + lax.axis_index("c")
    base = wid * RPT
    pltpu.sync_copy(srcl_h.at[wid], lsrc)
    pltpu.sync_copy(dstl_h.at[wid], ldst)
    pltpu.sync_copy(norml_h.at[wid], lnrm)
    pltpu.sync_copy(cnt_h.at[wid], csc_v)
    n = csc_v[pl.ds(0, 16)][0]
    nb = (n + KB - 1) // KB
    col0 = _iota16()
    cae = 2 * col0
    cbe = cae + 1
    zf = jnp.zeros((16,), jnp.float32)

    @plsc.parallel_loop(0, RPT)
    def _(r):
        ridx = jnp.full((16,), r, jnp.int32)
        for v in range(H // 16):
            plsc.store_scatter(acc, [ridx, col0 + v * 16], zf)

    def issue(b, rbuf, sem):
        pltpu.async_copy(xw_h.at[lsrc.at[pl.ds(b * KB, KB)]], rbuf, sem)

    def drain(rbuf, sem):
        pltpu.make_async_copy(xw_h.at[pl.ds(0, KB)], rbuf, sem).wait()

    def process(b, rbuf):
        off = b * KB

        def grp(g, carry):
            dlv = ldst[pl.ds(off + g * 16, 16)]
            nmv = lnrm[pl.ds(off + g * 16, 16)]
            e0 = g * 16
            for k in range(16):
                dl = _lane_take(dlv, k)
                nm = _lane_take(nmv, k)
                vals = []
                for c in range(H // 32):
                    abw = rbuf[e0 + k, pl.ds(c * 16, 16)]
                    ab = plsc.bitcast(abw, jnp.bfloat16)
                    av, bv = plsc.unpack(ab, format=plsc.PackFormat.INTERLEAVED)
                    vals.append((av * nm, bv * nm))
                for c in range(H // 32):
                    av, bv = vals[c]
                    plsc.addupdate_scatter(acc, [dl, col0 + c * 16], av)
                    plsc.addupdate_scatter(acc, [dl, col0 + c * 16 + H // 2], bv)
            return carry

        lax.fori_loop(0, KB // 16, grp, jnp.int32(0))

    bufs = ((rows0, sem0), (rows1, sem1), (rows2, sem2), (rows3, sem3))
    nd = len(bufs)

    for i in range(nd - 1):
        @pl.when(nb > i)
        def _(i=i):
            issue(i, bufs[i][0], bufs[i][1])

    def quad(t, carry):
        b0 = nd * t
        for j in range(nd):
            rbuf, sem = bufs[j]
            nbuf, nsem = bufs[(j + nd - 1) % nd]
            b = b0 + j

            @pl.when(b + nd - 1 < nb)
            def _(rbuf=nbuf, sem=nsem, b=b):
                issue(b + nd - 1, rbuf, sem)

            @pl.when(b < nb)
            def _(rbuf=rbuf, sem=sem, b=b):
                drain(rbuf, sem)
                process(b, rbuf)

        return carry

    lax.fori_loop(0, (nb + nd - 1) // nd, quad, jnp.int32(0))
    pltpu.sync_copy(acc, agg_h.at[pl.ds(base, RPT)])


def _sc_agg(xw, srcl, dstl, norml, cnt):
    mesh = plsc.VectorSubcoreMesh(core_axis_name="c", subcore_axis_name="s")
    f = pl.kernel(
        _sc_agg_body,
        out_type=jax.ShapeDtypeStruct((NP, H), jnp.float32),
        mesh=mesh,
        compiler_params=pltpu.CompilerParams(needs_layout_passes=False),
        scratch_types=[
            pltpu.VMEM((RPT, H), jnp.float32),
            pltpu.VMEM((KB, H // 2), jnp.int32),
            pltpu.VMEM((KB, H // 2), jnp.int32),
            pltpu.VMEM((KB, H // 2), jnp.int32),
            pltpu.VMEM((KB, H // 2), jnp.int32),
            pltpu.VMEM((CAP,), jnp.int32),
            pltpu.VMEM((CAP,), jnp.int32),
            pltpu.VMEM((CAP,), jnp.float32),
            pltpu.VMEM((16,), jnp.int32),
            pltpu.SemaphoreType.DMA,
            pltpu.SemaphoreType.DMA,
            pltpu.SemaphoreType.DMA,
            pltpu.SemaphoreType.DMA,
        ],
    )
    return f(xw, srcl, dstl, norml, cnt)


def _pack_bf16_words(x):
    xb = x.astype(jnp.bfloat16)
    lo = lax.bitcast_convert_type(xb[:, :H // 2], jnp.uint16).astype(jnp.uint32)
    hi = lax.bitcast_convert_type(xb[:, H // 2:], jnp.uint16).astype(jnp.uint32)
    return lax.bitcast_convert_type(lo | (hi << 16), jnp.int32)


def _tc_embed_body(pos_ref, we_ref, be_ref, w1_ref, xw_ref, xwb_ref):
    x = jnp.dot(pos_ref[...], we_ref[...], preferred_element_type=jnp.float32)
    x = jnp.maximum(x + be_ref[...], 0.0)
    xw = jnp.dot(x, w1_ref[...], preferred_element_type=jnp.float32)
    xw_ref[...] = xw
    xwb_ref[...] = _pack_bf16_words(xw)


def _tc_embed(pos_p, we_p, be, w1):
    return pl.pallas_call(
        _tc_embed_body,
        grid=(NBLK,),
        in_specs=[
            pl.BlockSpec((BLK, 128), lambda i: (i, 0)),
            pl.BlockSpec((128, H), lambda i: (0, 0)),
            pl.BlockSpec((1, H), lambda i: (0, 0)),
            pl.BlockSpec((H, H), lambda i: (0, 0)),
        ],
        out_specs=[
            pl.BlockSpec((BLK, H), lambda i: (i, 0)),
            pl.BlockSpec((BLK, H // 2), lambda i: (i, 0)),
        ],
        out_shape=[
            jax.ShapeDtypeStruct((NP, H), jnp.float32),
            jax.ShapeDtypeStruct((NP, H // 2), jnp.int32),
        ],
    )(pos_p, we_p, be, w1)


def _tc_post_body(agg_ref, xw_ref, dsq_ref, b_ref, y_ref, s1_ref, sq_ref):
    step = pl.program_id(0)
    y = agg_ref[...] + xw_ref[...] * dsq_ref[...] + b_ref[...]
    y = jnp.maximum(y, 0.0)
    rid = step * BLK + lax.broadcasted_iota(jnp.int32, (BLK, 1), 0)
    y = jnp.where(rid < N, y, 0.0)
    y_ref[...] = y
    s1 = jnp.sum(y, axis=0, keepdims=True)
    sq = jnp.sum(y * y, axis=0, keepdims=True)

    @pl.when(step == 0)
    def _():
        s1_ref[...] = s1
        sq_ref[...] = sq

    @pl.when(step != 0)
    def _():
        s1_ref[...] += s1
        sq_ref[...] += sq


def _tc_post(agg, xw, dsq_col, b):
    return pl.pallas_call(
        _tc_post_body,
        grid=(NBLK,),
        in_specs=[
            pl.BlockSpec((BLK, H), lambda i: (i, 0)),
            pl.BlockSpec((BLK, H), lambda i: (i, 0)),
            pl.BlockSpec((BLK, 1), lambda i: (i, 0)),
            pl.BlockSpec((1, H), lambda i: (0, 0)),
        ],
        out_specs=[
            pl.BlockSpec((BLK, H), lambda i: (i, 0)),
            pl.BlockSpec((1, H), lambda i: (0, 0)),
            pl.BlockSpec((1, H), lambda i: (0, 0)),
        ],
        out_shape=[
            jax.ShapeDtypeStruct((NP, H), jnp.float32),
            jax.ShapeDtypeStruct((1, H), jnp.float32),
            jax.ShapeDtypeStruct((1, H), jnp.float32),
        ],
    )(agg, xw, dsq_col, b)


def _pairnorm_stats(s1_ref, sq_ref):
    mu = s1_ref[...] * jnp.float32(1.0 / N)
    var = jnp.sum(sq_ref[...]) * jnp.float32(1.0 / N) - jnp.sum(mu * mu)
    inv_s = 1.0 / jnp.sqrt(1e-6 + var)
    return mu, inv_s


def _tc_mm_body(y_ref, s1_ref, sq_ref, w_ref, o_ref, ob_ref):
    mu, inv_s = _pairnorm_stats(s1_ref, sq_ref)
    h = (y_ref[...] - mu) * inv_s
    o = jnp.dot(h, w_ref[...], preferred_element_type=jnp.float32)
    o_ref[...] = o
    ob_ref[...] = _pack_bf16_words(o)


def _tc_mm(y, s1, sq, w):
    return pl.pallas_call(
        _tc_mm_body,
        grid=(NBLK,),
        in_specs=[
            pl.BlockSpec((BLK, H), lambda i: (i, 0)),
            pl.BlockSpec((1, H), lambda i: (0, 0)),
            pl.BlockSpec((1, H), lambda i: (0, 0)),
            pl.BlockSpec((H, H), lambda i: (0, 0)),
        ],
        out_specs=[
            pl.BlockSpec((BLK, H), lambda i: (i, 0)),
            pl.BlockSpec((BLK, H // 2), lambda i: (i, 0)),
        ],
        out_shape=[
            jax.ShapeDtypeStruct((NP, H), jnp.float32),
            jax.ShapeDtypeStruct((NP, H // 2), jnp.int32),
        ],
    )(y, s1, sq, w)


def _tc_pool_body(y_ref, bt_ref, s1_ref, sq_ref, wl1_ref, bl1_ref, wl2_ref,
                  bl2_ref, o_ref, hp_ref):
    step = pl.program_id(0)

    @pl.when(step == 0)
    def _():
        hp_ref[...] = jnp.full((G, H), -jnp.inf, jnp.float32)

    y = y_ref[...]
    bt = bt_ref[...]
    for g in range(G):
        yg = jnp.where(bt == g, y, -jnp.inf)
        mg = jnp.max(yg, axis=0, keepdims=True)
        hp_ref[pl.ds(g, 1), :] = jnp.maximum(hp_ref[pl.ds(g, 1), :], mg)

    @pl.when(step == NBLK - 1)
    def _():
        hp = hp_ref[...]
        fin = hp > -jnp.inf
        mu, inv_s = _pairnorm_stats(s1_ref, sq_ref)
        z = jnp.where(fin, (hp - mu) * inv_s, 0.0)
        h = jnp.maximum(
            jnp.dot(z, wl1_ref[...], preferred_element_type=jnp.float32)
            + bl1_ref[...], 0.0)
        o_ref[...] = (
            jnp.dot(h, wl2_ref[...], preferred_element_type=jnp.float32)
            + bl2_ref[...])


def _tc_pool_head(y, batch_col, s1, sq, wl1, bl1, wl2_p, bl2_p):
    return pl.pallas_call(
        _tc_pool_body,
        grid=(NBLK,),
        in_specs=[
            pl.BlockSpec((BLK, H), lambda i: (i, 0)),
            pl.BlockSpec((BLK, 1), lambda i: (i, 0)),
            pl.BlockSpec((1, H), lambda i: (0, 0)),
            pl.BlockSpec((1, H), lambda i: (0, 0)),
            pl.BlockSpec((H, H), lambda i: (0, 0)),
            pl.BlockSpec((1, H), lambda i: (0, 0)),
            pl.BlockSpec((H, 128), lambda i: (0, 0)),
            pl.BlockSpec((1, 128), lambda i: (0, 0)),
        ],
        out_specs=pl.BlockSpec((G, 128), lambda i: (0, 0)),
        out_shape=jax.ShapeDtypeStruct((G, 128), jnp.float32),
        scratch_shapes=[pltpu.VMEM((G, H), jnp.float32)],
    )(y, batch_col, s1, sq, wl1, bl1, wl2_p, bl2_p)


def kernel(pos, edge_index, edge_attr, batch, W_emb, b_emb, W1, b1, W2, b2,
           W3, b3, Wl1, bl1, Wl2, bl2):
    row = edge_index[0]
    col = edge_index[1]

    srcl, dstl, ewl, cnt, dinv, dsq = _sc_prep(row, col, edge_attr)
    norml = _sc_norm(dinv, srcl, dstl, ewl, cnt)
    dsq_col = dsq.reshape(NP, 1)

    pos_p = jnp.zeros((NP, 128), jnp.float32).at[:N, :2].set(pos)
    we_p = jnp.zeros((128, H), jnp.float32).at[:2, :].set(W_emb)
    batch_col = jnp.full((NP, 1), G, jnp.int32).at[:N, 0].set(batch)
    wl2_p = jnp.zeros((H, 128), jnp.float32).at[:, :2].set(Wl2)
    bl2_p = jnp.zeros((1, 128), jnp.float32).at[0, :2].set(bl2)

    xw, xwb = _tc_embed(pos_p, we_p, b_emb.reshape(1, H), W1)

    y = s1 = sq = None
    for b_cur, w_next in ((b1, W2), (b2, W3), (b3, None)):
        agg = _sc_agg(xwb, srcl, dstl, norml, cnt)
        y, s1, sq = _tc_post(agg, xw, dsq_col, b_cur.reshape(1, H))
        if w_next is not None:
            xw, xwb = _tc_mm(y, s1, sq, w_next)

    out = _tc_pool_head(y, batch_col, s1, sq, Wl1, bl1.reshape(1, H),
                        wl2_p, bl2_p)
    return out[:, :2]


# speedup vs baseline: 1.0853x; 1.0853x over previous
import jax
import jax.numpy as jnp
from jax import lax
from jax.experimental import pallas as pl
from jax.experimental.pallas import tpu as pltpu
from jax.experimental.pallas import tpu_sc as plsc

N, E, H, G = 10000, 160000, 256, 16
NT = 32
RPT = 320
NP = NT * RPT
CAP = 8064
KB = 64
ECH = 2000
NBLK = 10
BLK = NP // NBLK


def _iota16():
    return lax.iota(jnp.int32, 16)


def _lane_take(v, lane):
    idx = jnp.full((16,), lane, jnp.int32)
    return v.at[idx].get(mode="promise_in_bounds")


def _rsqrt16(x):
    i = plsc.bitcast(x, jnp.int32)
    y = plsc.bitcast(jnp.int32(0x5F3759DF) - (i >> 1), jnp.float32)
    for _ in range(4):
        y = y * (1.5 - 0.5 * x * y * y)
    return y


def _sc_prep_body(row_h, col_h, ew_h, srcl_h, dstl_h, ewl_h, cnt_h, dinv_h,
                  dsq_h, src_a, dst_a, ew_a, src_b, dst_b, ew_b, lsrc, ldst,
                  lew, deg_v, div_v, dsq_v, csc_v, sem_a, sem_b):
    wid = lax.axis_index("s") * 2 + lax.axis_index("c")
    base = wid * RPT
    zi = jnp.zeros((16,), jnp.int32)
    zf = jnp.zeros((16,), jnp.float32)
    nch = E // ECH

    def issue(b, bufs, sem):
        off = pl.multiple_of(b * ECH, 8)
        pltpu.async_copy(row_h.at[pl.ds(off, ECH)], bufs[0], sem)
        pltpu.async_copy(col_h.at[pl.ds(off, ECH)], bufs[1], sem)
        pltpu.async_copy(ew_h.at[pl.ds(off, ECH)], bufs[2], sem)

    def drain(bufs, sem):
        pltpu.make_async_copy(row_h.at[pl.ds(0, ECH)], bufs[0], sem).wait()
        pltpu.make_async_copy(row_h.at[pl.ds(0, ECH)], bufs[1], sem).wait()
        pltpu.make_async_copy(ew_h.at[pl.ds(0, ECH)], bufs[2], sem).wait()

    def compact(bufs, cnt):
        src_v, dst_v, ew_v = bufs
        GPB = 5

        def grp(i, cnt):
            gs = []
            for j in range(GPB):
                o = pl.ds((i * GPB + j) * 16, 16)
                sv = src_v[o]
                dv = dst_v[o]
                ev = ew_v[o]
                m = (dv >= base) & (dv < base + RPT)
                pos = plsc.cumsum(jnp.where(m, 1, 0).astype(jnp.int32))
                gs.append((sv, dv, ev, m, pos))
            for sv, dv, ev, m, pos in gs:
                offs = cnt + pos - 1
                mj = m & (offs < CAP - KB)
                plsc.store_scatter(lsrc, [offs], sv, mask=mj)
                plsc.store_scatter(ldst, [offs], dv - base, mask=mj)
                plsc.store_scatter(lew, [offs], ev, mask=mj)
                cnt = cnt + _lane_take(pos, 15)
            return cnt

        return lax.fori_loop(0, ECH // 16 // GPB, grp, cnt)

    bufs_a = (src_a, dst_a, ew_a)
    bufs_b = (src_b, dst_b, ew_b)
    issue(0, bufs_a, sem_a)

    def pair(p, cnt):
        b0 = 2 * p

        @pl.when(b0 + 1 < nch)
        def _():
            issue(b0 + 1, bufs_b, sem_b)

        drain(bufs_a, sem_a)
        cnt = compact(bufs_a, cnt)

        @pl.when(b0 + 2 < nch)
        def _():
            issue(b0 + 2, bufs_a, sem_a)

        drain(bufs_b, sem_b)
        cnt = compact(bufs_b, cnt)
        return cnt

    cnt_v16 = lax.fori_loop(0, nch // 2, pair, jnp.zeros((16,), jnp.int32))
    cnt = jnp.minimum(cnt_v16[0], jnp.int32(CAP - KB))

    for t in range(KB // 16):
        lsrc[pl.ds(cnt + t * 16, 16)] = zi
        ldst[pl.ds(cnt + t * 16, 16)] = zi
        lew[pl.ds(cnt + t * 16, 16)] = zf

    for t in range(RPT // 16):
        deg_v[pl.ds(t * 16, 16)] = zf

    def degg(g, carry):
        dlv = ldst[pl.ds(g * 16, 16)]
        ewv = lew[pl.ds(g * 16, 16)]
        for k in range(16):
            mk = _iota16() == k
            plsc.addupdate_scatter(deg_v, [dlv], ewv, mask=mk)
        return carry

    ng = (cnt + 15) // 16
    lax.fori_loop(0, ng, degg, jnp.int32(0))

    for t in range(RPT // 16):
        gidx = base + t * 16 + _iota16()
        d = deg_v[pl.ds(t * 16, 16)]
        d = d + jnp.where(gidx < N, 1.0, 0.0).astype(jnp.float32)
        r = _rsqrt16(jnp.maximum(d, 1e-12))
        r = jnp.where(d > 0, r, 0.0).astype(jnp.float32)
        div_v[pl.ds(t * 16, 16)] = r
        dsq_v[pl.ds(t * 16, 16)] = r * r

    csc_v[...] = jnp.full((16,), cnt, jnp.int32)
    pltpu.sync_copy(lsrc, srcl_h.at[wid])
    pltpu.sync_copy(ldst, dstl_h.at[wid])
    pltpu.sync_copy(lew, ewl_h.at[wid])
    pltpu.sync_copy(csc_v, cnt_h.at[wid])
    pltpu.sync_copy(div_v, dinv_h.at[pl.ds(base, RPT)])
    pltpu.sync_copy(dsq_v, dsq_h.at[pl.ds(base, RPT)])


def _sc_prep(row, col, ew):
    mesh = plsc.VectorSubcoreMesh(core_axis_name="c", subcore_axis_name="s")
    f = pl.kernel(
        _sc_prep_body,
        out_type=(
            jax.ShapeDtypeStruct((NT, CAP), jnp.int32),
            jax.ShapeDtypeStruct((NT, CAP), jnp.int32),
            jax.ShapeDtypeStruct((NT, CAP), jnp.float32),
            jax.ShapeDtypeStruct((NT, 16), jnp.int32),
            jax.ShapeDtypeStruct((NP,), jnp.float32),
            jax.ShapeDtypeStruct((NP,), jnp.float32),
        ),
        mesh=mesh,
        compiler_params=pltpu.CompilerParams(needs_layout_passes=False),
        scratch_types=[
            pltpu.VMEM((ECH,), jnp.int32),
            pltpu.VMEM((ECH,), jnp.int32),
            pltpu.VMEM((ECH,), jnp.float32),
            pltpu.VMEM((ECH,), jnp.int32),
            pltpu.VMEM((ECH,), jnp.int32),
            pltpu.VMEM((ECH,), jnp.float32),
            pltpu.VMEM((CAP,), jnp.int32),
            pltpu.VMEM((CAP,), jnp.int32),
            pltpu.VMEM((CAP,), jnp.float32),
            pltpu.VMEM((RPT,), jnp.float32),
            pltpu.VMEM((RPT,), jnp.float32),
            pltpu.VMEM((RPT,), jnp.float32),
            pltpu.VMEM((16,), jnp.int32),
            pltpu.SemaphoreType.DMA,
            pltpu.SemaphoreType.DMA,
        ],
    )
    return f(row, col, ew)


def _sc_norm_body(dinv_h, srcl_h, dstl_h, ewl_h, cnt_h, norml_h,
                  dinv_v, lsrc, ldst, lew, lnorm, csc_v):
    wid = lax.axis_index("s") * 2 + lax.axis_index("c")
    base = wid * RPT
    pltpu.sync_copy(dinv_h, dinv_v)
    pltpu.sync_copy(srcl_h.at[wid], lsrc)
    pltpu.sync_copy(dstl_h.at[wid], ldst)
    pltpu.sync_copy(ewl_h.at[wid], lew)
    pltpu.sync_copy(cnt_h.at[wid], csc_v)
    n = csc_v[pl.ds(0, 16)][0]
    ng = ((n + KB - 1) // KB) * (KB // 16)

    def grp(g, carry):
        sv = lsrc[pl.ds(g * 16, 16)]
        dlv = ldst[pl.ds(g * 16, 16)]
        ev = lew[pl.ds(g * 16, 16)]
        da = plsc.load_gather(dinv_v, [sv])
        db = plsc.load_gather(dinv_v, [dlv + base])
        lnorm[pl.ds(g * 16, 16)] = da * ev * db
        return carry

    lax.fori_loop(0, ng, grp, jnp.int32(0))
    pltpu.sync_copy(lnorm, norml_h.at[wid])


def _sc_norm(dinv, srcl, dstl, ewl, cnt):
    mesh = plsc.VectorSubcoreMesh(core_axis_name="c", subcore_axis_name="s")
    f = pl.kernel(
        _sc_norm_body,
        out_type=jax.ShapeDtypeStruct((NT, CAP), jnp.float32),
        mesh=mesh,
        compiler_params=pltpu.CompilerParams(needs_layout_passes=False),
        scratch_types=[
            pltpu.VMEM((NP,), jnp.float32),
            pltpu.VMEM((CAP,), jnp.int32),
            pltpu.VMEM((CAP,), jnp.int32),
            pltpu.VMEM((CAP,), jnp.float32),
            pltpu.VMEM((CAP,), jnp.float32),
            pltpu.VMEM((16,), jnp.int32),
        ],
    )
    return f(dinv, srcl, dstl, ewl, cnt)


def _sc_agg_body(xw_h, srcl_h, dstl_h, norml_h, cnt_h, agg_h,
                 acc, rows0, rows1, rows2, lsrc, ldst, lnrm, csc_v,
                 sem0, sem1, sem2):
    wid = lax.axis_index("s") * 2 + lax.axis_index("c")
    base = wid * RPT
    pltpu.sync_copy(srcl_h.at[wid], lsrc)
    pltpu.sync_copy(dstl_h.at[wid], ldst)
    pltpu.sync_copy(norml_h.at[wid], lnrm)
    pltpu.sync_copy(cnt_h.at[wid], csc_v)
    n = csc_v[pl.ds(0, 16)][0]
    nb = (n + KB - 1) // KB
    col0 = _iota16()
    cae = 2 * col0
    cbe = cae + 1
    zf = jnp.zeros((16,), jnp.float32)

    @plsc.parallel_loop(0, RPT)
    def _(r):
        ridx = jnp.full((16,), r, jnp.int32)
        for v in range(H // 16):
            plsc.store_scatter(acc, [ridx, col0 + v * 16], zf)

    def issue(b, rbuf, sem):
        pltpu.async_copy(xw_h.at[lsrc.at[pl.ds(b * KB, KB)]], rbuf, sem)

    def drain(rbuf, sem):
        pltpu.make_async_copy(xw_h.at[pl.ds(0, KB)], rbuf, sem).wait()

    def process(b, rbuf):
        off = b * KB

        def grp(g, carry):
            dlv = ldst[pl.ds(off + g * 16, 16)]
            nmv = lnrm[pl.ds(off + g * 16, 16)]
            e0 = g * 16
            for k in range(16):
                dl = _lane_take(dlv, k)
                nm = _lane_take(nmv, k)
                vals = []
                for c in range(H // 32):
                    abw = rbuf[e0 + k, pl.ds(c * 16, 16)]
                    ab = plsc.bitcast(abw, jnp.bfloat16)
                    av, bv = plsc.unpack(ab, format=plsc.PackFormat.INTERLEAVED)
                    vals.append((av * nm, bv * nm))
                for c in range(H // 32):
                    av, bv = vals[c]
                    plsc.addupdate_scatter(acc, [dl, col0 + c * 16], av)
                    plsc.addupdate_scatter(acc, [dl, col0 + c * 16 + H // 2], bv)
            return carry

        lax.fori_loop(0, KB // 16, grp, jnp.int32(0))

    bufs = ((rows0, sem0), (rows1, sem1), (rows2, sem2))
    nd = len(bufs)

    for i in range(nd - 1):
        @pl.when(nb > i)
        def _(i=i):
            issue(i, bufs[i][0], bufs[i][1])

    def quad(t, carry):
        b0 = nd * t
        for j in range(nd):
            rbuf, sem = bufs[j]
            nbuf, nsem = bufs[(j + nd - 1) % nd]
            b = b0 + j

            @pl.when(b + nd - 1 < nb)
            def _(rbuf=nbuf, sem=nsem, b=b):
                issue(b + nd - 1, rbuf, sem)

            @pl.when(b < nb)
            def _(rbuf=rbuf, sem=sem, b=b):
                drain(rbuf, sem)
                process(b, rbuf)

        return carry

    lax.fori_loop(0, (nb + nd - 1) // nd, quad, jnp.int32(0))
    pltpu.sync_copy(acc, agg_h.at[pl.ds(base, RPT)])


def _sc_agg(xw, srcl, dstl, norml, cnt):
    mesh = plsc.VectorSubcoreMesh(core_axis_name="c", subcore_axis_name="s")
    f = pl.kernel(
        _sc_agg_body,
        out_type=jax.ShapeDtypeStruct((NP, H), jnp.float32),
        mesh=mesh,
        compiler_params=pltpu.CompilerParams(needs_layout_passes=False),
        scratch_types=[
            pltpu.VMEM((RPT, H), jnp.float32),
            pltpu.VMEM((KB, H // 2), jnp.int32),
            pltpu.VMEM((KB, H // 2), jnp.int32),
            pltpu.VMEM((KB, H // 2), jnp.int32),
            pltpu.VMEM((CAP,), jnp.int32),
            pltpu.VMEM((CAP,), jnp.int32),
            pltpu.VMEM((CAP,), jnp.float32),
            pltpu.VMEM((16,), jnp.int32),
            pltpu.SemaphoreType.DMA,
            pltpu.SemaphoreType.DMA,
            pltpu.SemaphoreType.DMA,
        ],
    )
    return f(xw, srcl, dstl, norml, cnt)


def _pack_bf16_words(x):
    xb = x.astype(jnp.bfloat16)
    lo = lax.bitcast_convert_type(xb[:, :H // 2], jnp.uint16).astype(jnp.uint32)
    hi = lax.bitcast_convert_type(xb[:, H // 2:], jnp.uint16).astype(jnp.uint32)
    return lax.bitcast_convert_type(lo | (hi << 16), jnp.int32)


def _tc_embed_body(pos_ref, we_ref, be_ref, w1_ref, xw_ref, xwb_ref):
    x = jnp.dot(pos_ref[...], we_ref[...], preferred_element_type=jnp.float32)
    x = jnp.maximum(x + be_ref[...], 0.0)
    xw = jnp.dot(x, w1_ref[...], preferred_element_type=jnp.float32)
    xw_ref[...] = xw
    xwb_ref[...] = _pack_bf16_words(xw)


def _tc_embed(pos_p, we_p, be, w1):
    return pl.pallas_call(
        _tc_embed_body,
        grid=(NBLK,),
        in_specs=[
            pl.BlockSpec((BLK, 128), lambda i: (i, 0)),
            pl.BlockSpec((128, H), lambda i: (0, 0)),
            pl.BlockSpec((1, H), lambda i: (0, 0)),
            pl.BlockSpec((H, H), lambda i: (0, 0)),
        ],
        out_specs=[
            pl.BlockSpec((BLK, H), lambda i: (i, 0)),
            pl.BlockSpec((BLK, H // 2), lambda i: (i, 0)),
        ],
        out_shape=[
            jax.ShapeDtypeStruct((NP, H), jnp.float32),
            jax.ShapeDtypeStruct((NP, H // 2), jnp.int32),
        ],
    )(pos_p, we_p, be, w1)


def _tc_post_body(agg_ref, xw_ref, dsq_ref, b_ref, y_ref, s1_ref, sq_ref):
    step = pl.program_id(0)
    y = agg_ref[...] + xw_ref[...] * dsq_ref[...] + b_ref[...]
    y = jnp.maximum(y, 0.0)
    rid = step * BLK + lax.broadcasted_iota(jnp.int32, (BLK, 1), 0)
    y = jnp.where(rid < N, y, 0.0)
    y_ref[...] = y
    s1 = jnp.sum(y, axis=0, keepdims=True)
    sq = jnp.sum(y * y, axis=0, keepdims=True)

    @pl.when(step == 0)
    def _():
        s1_ref[...] = s1
        sq_ref[...] = sq

    @pl.when(step != 0)
    def _():
        s1_ref[...] += s1
        sq_ref[...] += sq


def _tc_post(agg, xw, dsq_col, b):
    return pl.pallas_call(
        _tc_post_body,
        grid=(NBLK,),
        in_specs=[
            pl.BlockSpec((BLK, H), lambda i: (i, 0)),
            pl.BlockSpec((BLK, H), lambda i: (i, 0)),
            pl.BlockSpec((BLK, 1), lambda i: (i, 0)),
            pl.BlockSpec((1, H), lambda i: (0, 0)),
        ],
        out_specs=[
            pl.BlockSpec((BLK, H), lambda i: (i, 0)),
            pl.BlockSpec((1, H), lambda i: (0, 0)),
            pl.BlockSpec((1, H), lambda i: (0, 0)),
        ],
        out_shape=[
            jax.ShapeDtypeStruct((NP, H), jnp.float32),
            jax.ShapeDtypeStruct((1, H), jnp.float32),
            jax.ShapeDtypeStruct((1, H), jnp.float32),
        ],
    )(agg, xw, dsq_col, b)


def _pairnorm_stats(s1_ref, sq_ref):
    mu = s1_ref[...] * jnp.float32(1.0 / N)
    var = jnp.sum(sq_ref[...]) * jnp.float32(1.0 / N) - jnp.sum(mu * mu)
    inv_s = 1.0 / jnp.sqrt(1e-6 + var)
    return mu, inv_s


def _tc_mm_body(y_ref, s1_ref, sq_ref, w_ref, o_ref, ob_ref):
    mu, inv_s = _pairnorm_stats(s1_ref, sq_ref)
    h = (y_ref[...] - mu) * inv_s
    o = jnp.dot(h, w_ref[...], preferred_element_type=jnp.float32)
    o_ref[...] = o
    ob_ref[...] = _pack_bf16_words(o)


def _tc_mm(y, s1, sq, w):
    return pl.pallas_call(
        _tc_mm_body,
        grid=(NBLK,),
        in_specs=[
            pl.BlockSpec((BLK, H), lambda i: (i, 0)),
            pl.BlockSpec((1, H), lambda i: (0, 0)),
            pl.BlockSpec((1, H), lambda i: (0, 0)),
            pl.BlockSpec((H, H), lambda i: (0, 0)),
        ],
        out_specs=[
            pl.BlockSpec((BLK, H), lambda i: (i, 0)),
            pl.BlockSpec((BLK, H // 2), lambda i: (i, 0)),
        ],
        out_shape=[
            jax.ShapeDtypeStruct((NP, H), jnp.float32),
            jax.ShapeDtypeStruct((NP, H // 2), jnp.int32),
        ],
    )(y, s1, sq, w)


def _tc_pool_body(y_ref, bt_ref, s1_ref, sq_ref, wl1_ref, bl1_ref, wl2_ref,
                  bl2_ref, o_ref, hp_ref):
    step = pl.program_id(0)

    @pl.when(step == 0)
    def _():
        hp_ref[...] = jnp.full((G, H), -jnp.inf, jnp.float32)

    y = y_ref[...]
    bt = bt_ref[...]
    for g in range(G):
        yg = jnp.where(bt == g, y, -jnp.inf)
        mg = jnp.max(yg, axis=0, keepdims=True)
        hp_ref[pl.ds(g, 1), :] = jnp.maximum(hp_ref[pl.ds(g, 1), :], mg)

    @pl.when(step == NBLK - 1)
    def _():
        hp = hp_ref[...]
        fin = hp > -jnp.inf
        mu, inv_s = _pairnorm_stats(s1_ref, sq_ref)
        z = jnp.where(fin, (hp - mu) * inv_s, 0.0)
        h = jnp.maximum(
            jnp.dot(z, wl1_ref[...], preferred_element_type=jnp.float32)
            + bl1_ref[...], 0.0)
        o_ref[...] = (
            jnp.dot(h, wl2_ref[...], preferred_element_type=jnp.float32)
            + bl2_ref[...])


def _tc_pool_head(y, batch_col, s1, sq, wl1, bl1, wl2_p, bl2_p):
    return pl.pallas_call(
        _tc_pool_body,
        grid=(NBLK,),
        in_specs=[
            pl.BlockSpec((BLK, H), lambda i: (i, 0)),
            pl.BlockSpec((BLK, 1), lambda i: (i, 0)),
            pl.BlockSpec((1, H), lambda i: (0, 0)),
            pl.BlockSpec((1, H), lambda i: (0, 0)),
            pl.BlockSpec((H, H), lambda i: (0, 0)),
            pl.BlockSpec((1, H), lambda i: (0, 0)),
            pl.BlockSpec((H, 128), lambda i: (0, 0)),
            pl.BlockSpec((1, 128), lambda i: (0, 0)),
        ],
        out_specs=pl.BlockSpec((G, 128), lambda i: (0, 0)),
        out_shape=jax.ShapeDtypeStruct((G, 128), jnp.float32),
        scratch_shapes=[pltpu.VMEM((G, H), jnp.float32)],
    )(y, batch_col, s1, sq, wl1, bl1, wl2_p, bl2_p)


def kernel(pos, edge_index, edge_attr, batch, W_emb, b_emb, W1, b1, W2, b2,
           W3, b3, Wl1, bl1, Wl2, bl2):
    row = edge_index[0]
    col = edge_index[1]

    srcl, dstl, ewl, cnt, dinv, dsq = _sc_prep(row, col, edge_attr)
    norml = _sc_norm(dinv, srcl, dstl, ewl, cnt)
    dsq_col = dsq.reshape(NP, 1)

    pos_p = jnp.zeros((NP, 128), jnp.float32).at[:N, :2].set(pos)
    we_p = jnp.zeros((128, H), jnp.float32).at[:2, :].set(W_emb)
    batch_col = jnp.full((NP, 1), G, jnp.int32).at[:N, 0].set(batch)
    wl2_p = jnp.zeros((H, 128), jnp.float32).at[:, :2].set(Wl2)
    bl2_p = jnp.zeros((1, 128), jnp.float32).at[0, :2].set(bl2)

    xw, xwb = _tc_embed(pos_p, we_p, b_emb.reshape(1, H), W1)

    y = s1 = sq = None
    for b_cur, w_next in ((b1, W2), (b2, W3), (b3, None)):
        agg = _sc_agg(xwb, srcl, dstl, norml, cnt)
        y, s1, sq = _tc_post(agg, xw, dsq_col, b_cur.reshape(1, H))
        if w_next is not None:
            xw, xwb = _tc_mm(y, s1, sq, w_next)

    out = _tc_pool_head(y, batch_col, s1, sq, Wl1, bl1.reshape(1, H),
                        wl2_p, bl2_p)
    return out[:, :2]
